# baseline (speedup 1.0000x reference)
import jax
import jax.numpy as jnp
from jax.experimental import pallas as pl


def _dot(a, b):
    return jnp.dot(a, b, preferred_element_type=jnp.float32)


def _cheb_layer(L, x, x16, W, b):
    t1 = _dot(L, x16)
    t2 = 2.0 * _dot(L, t1.astype(jnp.bfloat16)) - x
    out = _dot(x16, W[0])
    out = out + _dot(t1.astype(jnp.bfloat16), W[1])
    out = out + _dot(t2.astype(jnp.bfloat16), W[2])
    return out + b


def _net_kernel(x_ref, a_ref, w1_ref, b1_ref, w2_ref, b2_ref, w3_ref, b3_ref,
                o_ref):
    A = a_ref[0]
    deg = jnp.sum(A, axis=-1)
    dinv = jnp.where(deg > 0.0, jax.lax.rsqrt(jnp.where(deg > 0.0, deg, 1.0)),
                     0.0)
    L = (-(A * dinv[:, None] * dinv[None, :])).astype(jnp.bfloat16)

    x16 = x_ref[0]
    x = x16.astype(jnp.float32)
    h = jax.nn.relu(_cheb_layer(L, x, x16, w1_ref[...], b1_ref[...]))
    h16 = h.astype(jnp.bfloat16)
    h = jax.nn.relu(_cheb_layer(L, h, h16, w2_ref[...], b2_ref[...]))
    h16 = h.astype(jnp.bfloat16)
    o_ref[0] = _cheb_layer(L, h, h16, w3_ref[...], b3_ref[...])


def kernel(X, A, W1, b1, W2, b2, W3, b3):
    B, S, T, E = X.shape
    d_in = T * E
    d_out = W3.shape[-1]
    x = X.reshape(B, S, d_in).astype(jnp.bfloat16)
    W1 = W1.astype(jnp.bfloat16)
    W2 = W2.astype(jnp.bfloat16)
    W3 = W3.astype(jnp.bfloat16)

    def batch_spec(shape):
        return pl.BlockSpec((1,) + shape, lambda b: (b, 0, 0))

    def full_spec(arr):
        return pl.BlockSpec(arr.shape, lambda b: (0,) * arr.ndim)

    return pl.pallas_call(
        _net_kernel,
        grid=(B,),
        in_specs=[
            batch_spec((S, d_in)),
            batch_spec((S, S)),
            full_spec(W1), full_spec(b1),
            full_spec(W2), full_spec(b2),
            full_spec(W3), full_spec(b3),
        ],
        out_specs=batch_spec((S, d_out)),
        out_shape=jax.ShapeDtypeStruct((B, S, d_out), jnp.float32),
    )(x, A, W1, b1, W2, b2, W3, b3)


# speedup vs baseline: 1.1259x; 1.1259x over previous
import jax
import jax.numpy as jnp
from jax.experimental import pallas as pl

_GPB = 2


def _dot(a, b):
    return jnp.dot(a, b, preferred_element_type=jnp.float32)


def _cheb_layer(L, x, W, b):
    t1 = _dot(L, x)
    t2 = 2.0 * _dot(L, t1) - x
    out = _dot(x, W[0]) + _dot(t1, W[1]) + _dot(t2, W[2])
    return out + b


def _net_kernel(x_ref, a_ref, w1_ref, b1_ref, w2_ref, b2_ref, w3_ref, b3_ref,
                o_ref):
    for g in range(_GPB):
        A = a_ref[g]
        deg = jnp.sum(A, axis=-1)
        dinv = jnp.where(deg > 0.0,
                         jax.lax.rsqrt(jnp.where(deg > 0.0, deg, 1.0)), 0.0)
        L = -(A * dinv[:, None] * dinv[None, :])

        x = x_ref[g]
        h = jax.nn.relu(_cheb_layer(L, x, w1_ref[...], b1_ref[...]))
        h = jax.nn.relu(_cheb_layer(L, h, w2_ref[...], b2_ref[...]))
        o_ref[g] = _cheb_layer(L, h, w3_ref[...], b3_ref[...])


def kernel(X, A, W1, b1, W2, b2, W3, b3):
    B, S, T, E = X.shape
    d_in = T * E
    d_out = W3.shape[-1]
    x = X.reshape(B, S, d_in)

    def batch_spec(shape):
        return pl.BlockSpec((_GPB,) + shape, lambda b: (b, 0, 0))

    def full_spec(arr):
        return pl.BlockSpec(arr.shape, lambda b: (0,) * arr.ndim)

    return pl.pallas_call(
        _net_kernel,
        grid=(B // _GPB,),
        in_specs=[
            batch_spec((S, d_in)),
            batch_spec((S, S)),
            full_spec(W1), full_spec(b1),
            full_spec(W2), full_spec(b2),
            full_spec(W3), full_spec(b3),
        ],
        out_specs=batch_spec((S, d_out)),
        out_shape=jax.ShapeDtypeStruct((B, S, d_out), jnp.float32),
    )(x, A, W1, b1, W2, b2, W3, b3)


# speedup vs baseline: 1.1532x; 1.0242x over previous
import jax
import jax.numpy as jnp
from jax.experimental import pallas as pl

_GPB = 4


def _dot(a, b):
    return jnp.dot(a, b, preferred_element_type=jnp.float32)


def _cheb_layer(L, x, W, b):
    t1 = _dot(L, x)
    t2 = 2.0 * _dot(L, t1) - x
    out = _dot(x, W[0]) + _dot(t1, W[1]) + _dot(t2, W[2])
    return out + b


def _net_kernel(x_ref, a_ref, w1_ref, b1_ref, w2_ref, b2_ref, w3_ref, b3_ref,
                o_ref):
    for g in range(_GPB):
        A = a_ref[g]
        deg = jnp.sum(A, axis=-1)
        dinv = jnp.where(deg > 0.0,
                         jax.lax.rsqrt(jnp.where(deg > 0.0, deg, 1.0)), 0.0)
        L = -(A * dinv[:, None] * dinv[None, :])

        x = x_ref[g]
        h = jax.nn.relu(_cheb_layer(L, x, w1_ref[...], b1_ref[...]))
        h = jax.nn.relu(_cheb_layer(L, h, w2_ref[...], b2_ref[...]))
        o_ref[g] = _cheb_layer(L, h, w3_ref[...], b3_ref[...])


def kernel(X, A, W1, b1, W2, b2, W3, b3):
    B, S, T, E = X.shape
    d_in = T * E
    d_out = W3.shape[-1]
    x = X.reshape(B, S, d_in)

    def batch_spec(shape):
        return pl.BlockSpec((_GPB,) + shape, lambda b: (b, 0, 0))

    def full_spec(arr):
        return pl.BlockSpec(arr.shape, lambda b: (0,) * arr.ndim)

    return pl.pallas_call(
        _net_kernel,
        grid=(B // _GPB,),
        in_specs=[
            batch_spec((S, d_in)),
            batch_spec((S, S)),
            full_spec(W1), full_spec(b1),
            full_spec(W2), full_spec(b2),
            full_spec(W3), full_spec(b3),
        ],
        out_specs=batch_spec((S, d_out)),
        out_shape=jax.ShapeDtypeStruct((B, S, d_out), jnp.float32),
    )(x, A, W1, b1, W2, b2, W3, b3)
